# Initial kernel scaffold; baseline (speedup 1.0000x reference)
import jax
import jax.numpy as jnp
from jax.experimental import pallas as pl


def kernel(x, table):
    raise NotImplementedError("write your pallas kernel here")


# speedup vs baseline: 1.1022x; 1.1022x over previous
import functools

import numpy as np
import jax
import jax.numpy as jnp
from jax import lax
from jax.experimental import pallas as pl
from jax.experimental.pallas import tpu as pltpu
from jax.experimental.pallas import tpu_sc as plsc

_D = 1024
_B = 4
_S = 2048
_NC, _NS, _L = 2, 16, 16
_NW = _NC * _NS
_ROWS = _B * _S
_RPW = _ROWS // _NW
_CH = 32
_NCHUNK = _RPW // _CH
_GPR = _D // _L


def _pos_fix_np():
    pos = np.arange(_S, dtype=np.float32)
    fix = np.zeros((_S, _L), dtype=np.float32)
    fix[:, 0] = np.sin(pos)
    fix[:, 1] = np.cos(pos) - np.float32(1.0)
    return fix


_POS_FIX = _pos_fix_np()

_mesh = plsc.VectorSubcoreMesh(core_axis_name="c", subcore_axis_name="s")


@functools.partial(
    pl.kernel,
    out_type=jax.ShapeDtypeStruct((_ROWS, _D), jnp.float32),
    mesh=_mesh,
    scratch_types=[
        pltpu.VMEM((_RPW,), jnp.int32),
        pltpu.VMEM((_RPW, _L), jnp.float32),
        pltpu.VMEM((_CH, _D), jnp.float32),
        pltpu.VMEM((_CH, _D), jnp.float32),
        pltpu.SemaphoreType.DMA,
        pltpu.SemaphoreType.DMA,
    ],
)
def _emb_kernel(x_hbm, pos_hbm, table_hbm, out_hbm,
                idx_v, pos_v, buf_a, buf_b, sem_a, sem_b):
    wid = lax.axis_index("s") * _NC + lax.axis_index("c")
    base = wid * _RPW
    pbase = lax.rem(base, _S)

    pltpu.sync_copy(x_hbm.at[pl.ds(base, _RPW)], idx_v)
    pltpu.sync_copy(pos_hbm.at[pl.ds(pbase, _RPW)], pos_v)

    pattern = lax.rem(lax.iota(jnp.int32, 16), 2).astype(jnp.float32)

    bufs = (buf_a, buf_b)
    sems = (sem_a, sem_b)

    def start_gather(c):
        b = c % 2
        return pltpu.async_copy(
            table_hbm.at[idx_v.at[pl.ds(c * _CH, _CH)]], bufs[b], sems[b])

    cp = start_gather(0)
    for c in range(_NCHUNK):
        buf = bufs[c % 2]
        cp.wait()
        if c + 1 < _NCHUNK:
            cp = start_gather(c + 1)

        def add_pattern(i, _):
            r = i // _GPR
            g = i - r * _GPR
            buf[r, pl.ds(g * _L, _L)] += pattern
            return _

        lax.fori_loop(0, _CH * _GPR, add_pattern, None)

        def add_fix(r, _):
            buf[r, pl.ds(0, _L)] += pos_v[c * _CH + r]
            return _

        lax.fori_loop(0, _CH, add_fix, None)

        pltpu.sync_copy(buf, out_hbm.at[pl.ds(base + c * _CH, _CH)])


def kernel(x, table):
    xf = x.reshape(_ROWS)
    pos_fix = jnp.asarray(_POS_FIX)
    out = _emb_kernel(xf, pos_fix, table)
    return out.reshape(_B, _S, _D)


# speedup vs baseline: 2.1969x; 1.9933x over previous
import functools

import numpy as np
import jax
import jax.numpy as jnp
from jax import lax
from jax.experimental import pallas as pl
from jax.experimental.pallas import tpu as pltpu
from jax.experimental.pallas import tpu_sc as plsc

_D = 1024
_B = 4
_S = 2048
_NC, _NS, _L = 2, 16, 16
_NW = _NC * _NS
_ROWS = _B * _S
_RPW = _ROWS // _NW
_CH = 32
_NCHUNK = _RPW // _CH
_GPR = _D // _L


def _pos_fix_np():
    pos = np.arange(_S, dtype=np.float32)
    fix = np.zeros((_S, _L), dtype=np.float32)
    fix[:, 0] = np.sin(pos)
    fix[:, 1] = np.cos(pos) - np.float32(1.0)
    return fix


_POS_FIX = _pos_fix_np()

_mesh = plsc.VectorSubcoreMesh(core_axis_name="c", subcore_axis_name="s")


@functools.partial(
    pl.kernel,
    out_type=jax.ShapeDtypeStruct((_ROWS, _D), jnp.float32),
    mesh=_mesh,
    scratch_types=[
        pltpu.VMEM((_RPW,), jnp.int32),
        pltpu.VMEM((_RPW, _L), jnp.float32),
        pltpu.VMEM((_CH, _D), jnp.float32),
        pltpu.VMEM((_CH, _D), jnp.float32),
        pltpu.SemaphoreType.DMA,
        pltpu.SemaphoreType.DMA,
        pltpu.SemaphoreType.DMA,
        pltpu.SemaphoreType.DMA,
    ],
)
def _emb_kernel(x_hbm, pos_hbm, table_hbm, out_hbm,
                idx_v, pos_v, buf_a, buf_b, sg_a, sg_b, ss_a, ss_b):
    wid = lax.axis_index("s") * _NC + lax.axis_index("c")
    base = wid * _RPW
    pbase = lax.rem(base, _S)

    pltpu.sync_copy(x_hbm.at[pl.ds(base, _RPW)], idx_v)
    pltpu.sync_copy(pos_hbm.at[pl.ds(pbase, _RPW)], pos_v)

    pattern = lax.rem(lax.iota(jnp.int32, 16), 2).astype(jnp.float32)

    bufs = (buf_a, buf_b)
    g_sems = (sg_a, sg_b)
    s_sems = (ss_a, ss_b)

    def start_gather(c):
        b = c % 2
        return pltpu.async_copy(
            table_hbm.at[idx_v.at[pl.ds(c * _CH, _CH)]], bufs[b], g_sems[b])

    gathers = [None] * _NCHUNK
    stores = [None] * _NCHUNK
    gathers[0] = start_gather(0)
    for c in range(_NCHUNK):
        b = c % 2
        buf = bufs[b]
        gathers[c].wait()
        if c + 1 < _NCHUNK:
            if c >= 1:
                stores[c - 1].wait()
            gathers[c + 1] = start_gather(c + 1)

        def row_body(r, _):
            for g in range(_GPR):
                buf[r, pl.ds(g * _L, _L)] += pattern
            buf[r, pl.ds(0, _L)] += pos_v[c * _CH + r]
            return _

        lax.fori_loop(0, _CH, row_body, None)

        stores[c] = pltpu.async_copy(
            buf, out_hbm.at[pl.ds(base + c * _CH, _CH)], s_sems[b])
    stores[_NCHUNK - 2].wait()
    stores[_NCHUNK - 1].wait()


def kernel(x, table):
    xf = x.reshape(_ROWS)
    pos_fix = jnp.asarray(_POS_FIX)
    out = _emb_kernel(xf, pos_fix, table)
    return out.reshape(_B, _S, _D)


# speedup vs baseline: 2.2596x; 1.0285x over previous
import functools

import numpy as np
import jax
import jax.numpy as jnp
from jax import lax
from jax.experimental import pallas as pl
from jax.experimental.pallas import tpu as pltpu
from jax.experimental.pallas import tpu_sc as plsc

_D = 1024
_B = 4
_S = 2048
_NC, _NS, _L = 2, 16, 16
_NW = _NC * _NS
_ROWS = _B * _S
_RPW = _ROWS // _NW
_CH = 32
_NCHUNK = _RPW // _CH
_GPR = _D // _L


def _pos_fix_np():
    pos = np.arange(_S, dtype=np.float32)
    fix = np.zeros((_S, _L), dtype=np.float32)
    fix[:, 0] = np.sin(pos)
    fix[:, 1] = np.cos(pos) - np.float32(1.0)
    return fix.reshape(-1)


_POS_FIX = _pos_fix_np()

_mesh = plsc.VectorSubcoreMesh(core_axis_name="c", subcore_axis_name="s")


@functools.partial(
    pl.kernel,
    out_type=jax.ShapeDtypeStruct((_B, _S, _D), jnp.float32),
    mesh=_mesh,
    scratch_types=[
        pltpu.VMEM((_RPW,), jnp.int32),
        pltpu.VMEM((_RPW * _L,), jnp.float32),
        pltpu.VMEM((_CH, _D), jnp.float32),
        pltpu.VMEM((_CH, _D), jnp.float32),
        pltpu.SemaphoreType.DMA,
        pltpu.SemaphoreType.DMA,
        pltpu.SemaphoreType.DMA,
        pltpu.SemaphoreType.DMA,
    ],
)
def _emb_kernel(x_hbm, pos_hbm, table_hbm, out_hbm,
                idx_v, pos_v, buf_a, buf_b, sg_a, sg_b, ss_a, ss_b):
    wid = lax.axis_index("s") * _NC + lax.axis_index("c")
    bi = wid // (_S // _RPW)
    pbase = lax.rem(wid, _S // _RPW) * _RPW

    pltpu.sync_copy(x_hbm.at[bi, pl.ds(pbase, _RPW)], idx_v)
    pltpu.sync_copy(pos_hbm.at[pl.ds(pbase * _L, _RPW * _L)], pos_v)

    pattern = lax.rem(lax.iota(jnp.int32, 16), 2).astype(jnp.float32)

    bufs = (buf_a, buf_b)
    g_sems = (sg_a, sg_b)
    s_sems = (ss_a, ss_b)

    def start_gather(c):
        b = c % 2
        return pltpu.async_copy(
            table_hbm.at[idx_v.at[pl.ds(c * _CH, _CH)]], bufs[b], g_sems[b])

    gathers = [None] * _NCHUNK
    stores = [None] * _NCHUNK
    gathers[0] = start_gather(0)
    for c in range(_NCHUNK):
        b = c % 2
        buf = bufs[b]
        gathers[c].wait()
        if c + 1 < _NCHUNK:
            if c >= 1:
                stores[c - 1].wait()
            gathers[c + 1] = start_gather(c + 1)

        def row_body(r, _):
            for g in range(_GPR):
                buf[r, pl.ds(g * _L, _L)] += pattern
            buf[r, pl.ds(0, _L)] += pos_v[pl.ds((c * _CH + r) * _L, _L)]
            return _

        lax.fori_loop(0, _CH, row_body, None)

        stores[c] = pltpu.async_copy(
            buf, out_hbm.at[bi, pl.ds(pbase + c * _CH, _CH)], s_sems[b])
    stores[_NCHUNK - 2].wait()
    stores[_NCHUNK - 1].wait()


def kernel(x, table):
    pos_fix = jnp.asarray(_POS_FIX)
    return _emb_kernel(x, pos_fix, table)
